# Initial kernel scaffold; baseline (speedup 1.0000x reference)
import jax
import jax.numpy as jnp
from jax.experimental import pallas as pl


def kernel(positive_sim, negative_sim):
    raise NotImplementedError("write your pallas kernel here")


# speedup vs baseline: 2.4264x; 2.4264x over previous
import jax
import jax.numpy as jnp
from jax.experimental import pallas as pl
from jax.experimental.pallas import tpu as pltpu

TEMP_ = 0.05
K_ = 32
ROWS_PER_BLOCK = 8
N_COLS = 32768
N_ROWS = 128


def _body(pos_ref, neg_ref, out_ref):
    s = neg_ref[...] - pos_ref[...]
    o = jnp.exp(TEMP_ * s)
    d = (o - 1.0) * (o - 1.0)
    db = jax.lax.bitcast_convert_type(d, jnp.int32)

    def bit_step(i, t):
        b = 30 - i
        cand = t | (1 << b)
        cnt = jnp.sum((db >= cand).astype(jnp.int32), axis=1, keepdims=True)
        return jnp.where(cnt >= K_, cand, t)

    t0 = jnp.zeros((ROWS_PER_BLOCK, 1), jnp.int32)
    t = jax.lax.fori_loop(0, 31, bit_step, t0)

    gt = db > t
    eq = db == t
    c_gt = jnp.sum(gt.astype(jnp.float32), axis=1)
    n_eq = jnp.sum(eq.astype(jnp.float32), axis=1)
    sum_gt = jnp.sum(jnp.where(gt, o, 0.0), axis=1)
    sum_eq = jnp.sum(jnp.where(eq, o, 0.0), axis=1)
    row_sum = sum_gt + (K_ - c_gt) / n_eq * sum_eq

    @pl.when(pl.program_id(0) == 0)
    def _():
        out_ref[...] = jnp.zeros_like(out_ref)

    out_ref[...] += jnp.sum(row_sum).reshape(1, 1)


def kernel(positive_sim, negative_sim):
    grid = (N_ROWS // ROWS_PER_BLOCK,)
    total = pl.pallas_call(
        _body,
        grid=grid,
        in_specs=[
            pl.BlockSpec((ROWS_PER_BLOCK, N_COLS), lambda i: (i, 0)),
            pl.BlockSpec((ROWS_PER_BLOCK, N_COLS), lambda i: (i, 0)),
        ],
        out_specs=pl.BlockSpec((1, 1), lambda i: (0, 0)),
        out_shape=jax.ShapeDtypeStruct((1, 1), jnp.float32),
    )(positive_sim, negative_sim)
    return total[0, 0] / jnp.float32(N_ROWS * K_)
